# baseline (speedup 1.0000x reference)
import functools

import jax
import jax.numpy as jnp
from jax import lax
from jax.experimental import pallas as pl
from jax.experimental.pallas import tpu as pltpu
from jax.experimental.pallas import tpu_sc as plsc

M = 32768
S = 2048
D = 128
T = 16
E = M // T
NV = E // 16
R = 256
KEEP = M - S
NW = 32
ROWS_W = KEEP // NW
IN_W = S // NW
CH = 96
NCH = ROWS_W // CH


def _iota16():
    return lax.iota(jnp.int32, 16)


def _take16(x, idx):
    return lax.gather(
        x, idx[:, None],
        dimension_numbers=lax.GatherDimensionNumbers(
            offset_dims=(), collapsed_slice_dims=(0,), start_index_map=(0,)),
        slice_sizes=(1,),
        mode=lax.GatherScatterMode.PROMISE_IN_BOUNDS)


def _vreg_rank(k16, shift):
    iota = _iota16()
    d = jnp.bitwise_and(jnp.right_shift(k16, shift), 0xFF)
    if shift == 24:
        d = jnp.bitwise_xor(d, 0x80)
    c = d * 16 + iota
    sc, sl = plsc.sort_key_val(c, iota)
    sd = jnp.right_shift(sc, 4)
    prev = _take16(sd, jnp.maximum(iota - 1, 0))
    is_new = jnp.logical_or(iota == 0, sd != prev)
    seg_start = plsc.cummax(jnp.where(is_new, iota, 0))
    rank = iota - seg_start
    nxt = _take16(is_new.astype(jnp.int32), jnp.minimum(iota + 1, 15))
    is_last = jnp.logical_or(iota == 15, nxt != 0)
    return sd, sl, rank, is_last


def _body(mem_hbm, inp_hbm, idx_hbm, prio_hbm, out_hbm,
          kA, vA, kB, vB, invp, histgrid,
          kf32, kbuf, vbuf1, vbuf2, dbuf, hist, histloc, offs,
          rowbuf, idxb, idxc, sem):
    c_id = lax.axis_index("c")
    s_id = lax.axis_index("s")
    t = s_id
    w = c_id * 16 + s_id
    iota = _iota16()

    def zero_hist():
        def zb(r, _):
            hist[pl.ds(r * 16, 16)] = jnp.zeros(16, jnp.int32)
            return 0
        lax.fori_loop(0, R // 16, zb, 0)

    def hist_loop(shift):
        def hb(i, _):
            k16 = kbuf[pl.ds(i * 16, 16)]
            sd, _sl, rank, is_last = _vreg_rank(k16, shift)
            plsc.addupdate_scatter(hist, [sd], rank + 1, mask=is_last)
            return 0
        lax.fori_loop(0, NV, hb, 0)

    def offsets_loop():
        def ob(dd, rt):
            cnts = plsc.load_gather(histloc, [iota, jnp.full(16, dd, jnp.int32)])
            incl = plsc.cumsum(cnts)
            excl = incl - cnts
            plsc.store_scatter(offs, [jnp.full(16, dd, jnp.int32)],
                               rt + excl, mask=iota == t)
            return rt + jnp.sum(cnts)
        lax.fori_loop(0, R, ob, jnp.int32(0))

    def place_loop(shift):
        def pb(i, _):
            k16 = kbuf[pl.ds(i * 16, 16)]
            sd, sl, rank, is_last = _vreg_rank(k16, shift)
            offv = plsc.load_gather(offs, [sd])
            destv = offv + rank
            plsc.store_scatter(offs, [sd], destv + 1, mask=is_last)
            row = jnp.full(16, i // 8, jnp.int32)
            col = (i % 8) * 16 + sl
            plsc.store_scatter(dbuf, [row, col], destv)
            return 0
        lax.fori_loop(0, NV, pb, 0)

    def scatter_kv(dst_k, dst_v):
        descs = []
        for r in range(16):
            descs.append(pltpu.async_copy(
                kbuf.at[pl.ds(r * 128, 128)], dst_k.at[dbuf.at[r]], sem))
            descs.append(pltpu.async_copy(
                vbuf1.at[pl.ds(r * 128, 128)], dst_v.at[dbuf.at[r]], sem))
        for dsc in descs:
            dsc.wait()

    @pl.when(t < 15)
    def _():
        pltpu.sync_copy(idx_hbm.at[pl.ds(S + t * E, E)], kf32)

    @pl.when(t == 15)
    def _():
        pltpu.sync_copy(prio_hbm.at[pl.ds(0, E)], kf32)

    def init_b(i, _):
        kf = kf32[pl.ds(i * 16, 16)]
        u = lax.bitcast_convert_type(kf, jnp.int32)
        kbuf[pl.ds(i * 16, 16)] = jnp.bitwise_xor(
            u, jnp.bitwise_and(jnp.right_shift(u, 31), 0x7FFFFFFF))
        vbuf1[pl.ds(i * 16, 16)] = t * E + i * 16 + iota
        return 0
    lax.fori_loop(0, NV, init_b, 0)

    zero_hist()
    hist_loop(0)
    pltpu.sync_copy(hist, histgrid.at[t])
    plsc.subcore_barrier()
    pltpu.sync_copy(histgrid, histloc)
    offsets_loop()
    place_loop(0)
    scatter_kv(kA, vA)
    plsc.subcore_barrier()

    for shift, (src_k, src_v), (dst_k, dst_v) in (
            (8, (kA, vA), (kB, vB)),
            (16, (kB, vB), (kA, vA))):
        pltpu.sync_copy(src_k.at[pl.ds(t * E, E)], kbuf)
        pltpu.sync_copy(src_v.at[pl.ds(t * E, E)], vbuf1)
        zero_hist()
        hist_loop(shift)
        pltpu.sync_copy(hist, histgrid.at[t])
        plsc.subcore_barrier()
        pltpu.sync_copy(histgrid, histloc)
        offsets_loop()
        place_loop(shift)
        scatter_kv(dst_k, dst_v)
        plsc.subcore_barrier()

    pltpu.sync_copy(kA.at[pl.ds(t * E, E)], kbuf)
    for r in range(16):
        pltpu.sync_copy(vA.at[pl.ds(t * E + r * 128, 128)], vbuf2.at[r])
    zero_hist()
    hist_loop(24)
    pltpu.sync_copy(hist, histgrid.at[t])
    plsc.subcore_barrier()
    pltpu.sync_copy(histgrid, histloc)
    offsets_loop()
    place_loop(24)
    descs = [pltpu.async_copy(dbuf.at[r], invp.at[vbuf2.at[r]], sem)
             for r in range(16)]
    for dsc in descs:
        dsc.wait()
    plsc.subcore_barrier()

    for j in range(NCH):
        pltpu.sync_copy(mem_hbm.at[pl.ds(S + w * ROWS_W + j * CH, CH)], rowbuf)
        pltpu.sync_copy(invp.at[pl.ds(w * ROWS_W + j * CH, CH)], idxb.at[0])
        pltpu.sync_copy(rowbuf, out_hbm.at[idxb.at[0]])
    pltpu.sync_copy(inp_hbm.at[pl.ds(w * IN_W, IN_W)], rowbuf.at[pl.ds(0, IN_W)])
    pltpu.sync_copy(invp.at[pl.ds(KEEP + w * IN_W, IN_W)], idxc.at[0])
    pltpu.sync_copy(rowbuf.at[pl.ds(0, IN_W)], out_hbm.at[idxc.at[0]])


@jax.jit
def _queue_update(mem2, inp2, idxf, priof):
    mesh = plsc.VectorSubcoreMesh(core_axis_name="c", subcore_axis_name="s")
    f = functools.partial(
        pl.kernel,
        out_type=jax.ShapeDtypeStruct((M, D), jnp.float32),
        mesh=mesh,
        compiler_params=pltpu.CompilerParams(needs_layout_passes=False),
        scratch_types=[
            pltpu.VMEM_SHARED((M,), jnp.int32),
            pltpu.VMEM_SHARED((M,), jnp.int32),
            pltpu.VMEM_SHARED((M,), jnp.int32),
            pltpu.VMEM_SHARED((M,), jnp.int32),
            pltpu.VMEM_SHARED((M,), jnp.int32),
            pltpu.VMEM_SHARED((T, R), jnp.int32),
            pltpu.VMEM((E,), jnp.float32),
            pltpu.VMEM((E,), jnp.int32),
            pltpu.VMEM((E,), jnp.int32),
            pltpu.VMEM((16, 128), jnp.int32),
            pltpu.VMEM((16, 128), jnp.int32),
            pltpu.VMEM((R,), jnp.int32),
            pltpu.VMEM((T, R), jnp.int32),
            pltpu.VMEM((R,), jnp.int32),
            pltpu.VMEM((CH, D), jnp.float32),
            pltpu.VMEM((1, CH), jnp.int32),
            pltpu.VMEM((1, IN_W), jnp.int32),
            pltpu.SemaphoreType.DMA,
        ],
    )(_body)
    return f(mem2, inp2, idxf, priof)


def kernel(inputs, priority, memory, index):
    B, S_, D_ = inputs.shape
    M_ = memory.shape[1]
    out = _queue_update(
        memory.reshape(M_, D_),
        inputs.reshape(S_, D_),
        index.reshape(M_),
        priority.reshape(S_),
    )
    return out.reshape(B, M_, D_)
